# Initial kernel scaffold; baseline (speedup 1.0000x reference)
import jax
import jax.numpy as jnp
from jax.experimental import pallas as pl


def kernel(hidden_states, gate_w):
    raise NotImplementedError("write your pallas kernel here")


# speedup vs baseline: 1.0970x; 1.0970x over previous
import functools

import jax
import jax.numpy as jnp
from jax import lax
from jax.experimental import pallas as pl
from jax.experimental.pallas import tpu as pltpu

_E = 16
_K = 2


def _tc_body(hs_ref, gate_ref, logits_ref, mask_ref, final_ref):
    x = hs_ref[...]
    g = gate_ref[...]
    logits = lax.dot_general(
        g, x, (((1,), (1,)), ((), ())),
        preferred_element_type=jnp.float32,
        precision=lax.Precision.DEFAULT,
    )
    bt = logits.shape[1]
    e_iota = lax.broadcasted_iota(jnp.int32, (_E, bt), 0)
    m1 = jnp.max(logits, axis=0, keepdims=True)
    idx1 = jnp.min(jnp.where(logits == m1, e_iota, _E), axis=0, keepdims=True)
    sel1 = e_iota == idx1
    neg = jnp.where(sel1, -jnp.inf, logits)
    m2 = jnp.max(neg, axis=0, keepdims=True)
    idx2 = jnp.min(jnp.where(neg == m2, e_iota, _E), axis=0, keepdims=True)
    sel2 = e_iota == idx2
    logits_ref[...] = logits
    mask_ref[:, 0, :] = sel1.astype(jnp.int32)
    mask_ref[:, 1, :] = sel2.astype(jnp.int32)
    final_ref[...] = jnp.zeros_like(final_ref)


def kernel(hidden_states, gate_w):
    batch, seq, d = hidden_states.shape
    t = batch * seq
    hs = hidden_states.reshape(t, d)
    bt = 512
    grid = t // bt
    logits_t, mask, final = pl.pallas_call(
        _tc_body,
        grid=(grid,),
        in_specs=[
            pl.BlockSpec((bt, d), lambda i: (i, 0)),
            pl.BlockSpec((_E, d), lambda i: (0, 0)),
        ],
        out_specs=[
            pl.BlockSpec((_E, bt), lambda i: (0, i)),
            pl.BlockSpec((_E, _K, bt), lambda i: (0, 0, i)),
            pl.BlockSpec((bt, d), lambda i: (i, 0)),
        ],
        out_shape=[
            jax.ShapeDtypeStruct((_E, t), jnp.float32),
            jax.ShapeDtypeStruct((_E, _K, t), jnp.int32),
            jax.ShapeDtypeStruct((t, d), jnp.float32),
        ],
    )(hs, gate_w)
    del logits_t
    return final, mask
